# Initial kernel scaffold; baseline (speedup 1.0000x reference)
import functools

import jax
import jax.numpy as jnp
from jax import lax
from jax.experimental import pallas as pl
from jax.experimental.pallas import tpu as pltpu
from jax.experimental.pallas import tpu_sc as plsc

NN = 10000
NE = 5000
NI = 320000
HID = 128

NC = 2
NS = 16
NW = NC * NS
PER_W = NI // NW
G = 80
NCH = PER_W // G

NE_H = 5008
NE_PAD = 5120
NN_PAD = 10240

_MESH = plsc.VectorSubcoreMesh(
    core_axis_name="c", subcore_axis_name="s", num_cores=NC, num_subcores=NS)

_f32 = jnp.float32
_i32 = jnp.int32


def _zero_1d(ref, n16):
  z = jnp.zeros((16,), _f32)
  def body(i, _):
    ref[pl.ds(i * 16, 16)] = z
    return 0
  lax.fori_loop(0, n16, body, 0)


@functools.partial(
    pl.kernel,
    out_type=(jax.ShapeDtypeStruct((NW, NN), _f32),
              jax.ShapeDtypeStruct((NW, NE_H), _f32)),
    mesh=_MESH,
    scratch_types=[
        pltpu.VMEM((PER_W,), _i32),
        pltpu.VMEM((PER_W,), _i32),
        pltpu.VMEM((NN,), _f32),
        pltpu.VMEM((NE_H,), _f32),
    ])
def _sc_counts(nidx_hbm, eidx_hbm, ncnt_out, ecnt_out,
               nidx_v, eidx_v, ncnt_v, ecnt_v):
  cid = lax.axis_index("c")
  sid = lax.axis_index("s")
  wid = cid * NS + sid
  base = wid * PER_W
  pltpu.sync_copy(nidx_hbm.at[pl.ds(base, PER_W)], nidx_v)
  pltpu.sync_copy(eidx_hbm.at[pl.ds(base, PER_W)], eidx_v)
  _zero_1d(ncnt_v, NN // 16)
  _zero_1d(ecnt_v, NE_H // 16)
  ones = jnp.ones((16,), _f32)
  def body(i, _):
    ni = nidx_v[pl.ds(i * 16, 16)]
    ei = eidx_v[pl.ds(i * 16, 16)]
    plsc.addupdate_scatter(ncnt_v, [ni], ones)
    plsc.addupdate_scatter(ecnt_v, [ei], ones)
    return 0
  lax.fori_loop(0, PER_W // 16, body, 0)
  pltpu.sync_copy(ncnt_v, ncnt_out.at[wid])
  pltpu.sync_copy(ecnt_v, ecnt_out.at[wid])


@functools.partial(
    pl.kernel,
    out_type=(jax.ShapeDtypeStruct((NW, NN), _f32),
              jax.ShapeDtypeStruct((NW, NE_H), _f32)),
    mesh=_MESH,
    scratch_types=[
        pltpu.VMEM((PER_W,), _i32),
        pltpu.VMEM((PER_W,), _i32),
        pltpu.VMEM((NN,), _f32),
        pltpu.VMEM((NE,), _f32),
        pltpu.VMEM((NN,), _f32),
        pltpu.VMEM((NE_H,), _f32),
    ])
def _sc_denoms(nidx_hbm, eidx_hbm, ncard_hbm, ecard_hbm, ndnm_out, ednm_out,
               nidx_v, eidx_v, ncard_v, ecard_v, ndnm_v, ednm_v):
  cid = lax.axis_index("c")
  sid = lax.axis_index("s")
  wid = cid * NS + sid
  base = wid * PER_W
  pltpu.sync_copy(nidx_hbm.at[pl.ds(base, PER_W)], nidx_v)
  pltpu.sync_copy(eidx_hbm.at[pl.ds(base, PER_W)], eidx_v)
  pltpu.sync_copy(ncard_hbm, ncard_v)
  pltpu.sync_copy(ecard_hbm, ecard_v)
  _zero_1d(ndnm_v, NN // 16)
  _zero_1d(ednm_v, NE_H // 16)
  def body(i, _):
    ni = nidx_v[pl.ds(i * 16, 16)]
    ei = eidx_v[pl.ds(i * 16, 16)]
    nc = plsc.load_gather(ncard_v, [ni])
    ec = plsc.load_gather(ecard_v, [ei])
    plsc.addupdate_scatter(ednm_v, [ei], nc)
    plsc.addupdate_scatter(ndnm_v, [ni], ec)
    return 0
  lax.fori_loop(0, PER_W // 16, body, 0)
  pltpu.sync_copy(ndnm_v, ndnm_out.at[wid])
  pltpu.sync_copy(ednm_v, ednm_out.at[wid])


def _make_rowpass(s_pad):
  zr = s_pad // NS
  assert zr % G == 0

  @functools.partial(
      pl.kernel,
      out_type=jax.ShapeDtypeStruct((NC, s_pad, HID), _f32),
      mesh=_MESH,
      scratch_types=[
          pltpu.VMEM((G,), _i32),
          pltpu.VMEM((G,), _i32),
          pltpu.VMEM((G, HID), _f32),
          pltpu.VMEM_SHARED((s_pad, HID), _f32),
          pltpu.SemaphoreType.DMA,
      ])
  def rowpass(table_hbm, gidx_hbm, sidx_hbm, out_hbm,
              gi_v, si_v, rows_v, acc_sh, sem):
    cid = lax.axis_index("c")
    sid = lax.axis_index("s")
    wid = cid * NS + sid
    z = jnp.zeros((16,), _f32)
    def zb(r, _):
      for j in range(HID // 16):
        rows_v[r, pl.ds(j * 16, 16)] = z
      return 0
    lax.fori_loop(0, G, zb, 0)
    for k in range(zr // G):
      pltpu.sync_copy(rows_v, acc_sh.at[pl.ds(sid * zr + k * G, G)])
    plsc.subcore_barrier()
    base = wid * PER_W
    def body(j, _):
      pltpu.sync_copy(gidx_hbm.at[pl.ds(base + j * G, G)], gi_v)
      pltpu.sync_copy(sidx_hbm.at[pl.ds(base + j * G, G)], si_v)
      pltpu.async_copy(table_hbm.at[gi_v], rows_v, sem).wait()
      pltpu.sync_copy(rows_v, acc_sh.at[si_v], add=True)
      return 0
    lax.fori_loop(0, NCH, body, 0)
    plsc.subcore_barrier()
    for k in range(zr // G):
      r0 = sid * zr + k * G
      pltpu.sync_copy(acc_sh.at[pl.ds(r0, G)], rows_v)
      pltpu.sync_copy(rows_v, out_hbm.at[cid, pl.ds(r0, G)])

  return rowpass


_rowpass_edge = _make_rowpass(NE_PAD)
_rowpass_node = _make_rowpass(NN_PAD)


def _tc_prep_body(ncnt_ref, ecnt_ref, x0_ref, w01_ref,
                  ncard_ref, ecard_ref, msg_ref):
  ncnt = jnp.sum(ncnt_ref[...], axis=0)
  ecnt = jnp.sum(ecnt_ref[...], axis=0)[:NE]
  ncard = lax.rsqrt(ncnt)
  r = lax.rsqrt(ecnt)
  ecard = r * r * r
  ncard_ref[...] = ncard
  ecard_ref[...] = ecard
  m = jnp.dot(x0_ref[...], w01_ref[...], preferred_element_type=_f32)
  msg_ref[...] = ncard[:, None] * m


def _tc_prep(ncnt_p, ecnt_p, x0, w01):
  return pl.pallas_call(
      _tc_prep_body,
      out_shape=(jax.ShapeDtypeStruct((NN,), _f32),
                 jax.ShapeDtypeStruct((NE,), _f32),
                 jax.ShapeDtypeStruct((NN, HID), _f32)),
  )(ncnt_p, ecnt_p, x0, w01)


def _tc_edge_body(part_ref, ednm_ref, ecard_ref, b1_ref, w10_ref, msg_ref):
  agg = part_ref[0, :NE, :] + part_ref[1, :NE, :]
  left1 = 1.0 / jnp.sum(ednm_ref[...], axis=0)[:NE]
  x1 = jnp.maximum(left1[:, None] * agg + b1_ref[...][None, :], 0.0)
  m = jnp.dot(x1, w10_ref[...], preferred_element_type=_f32)
  msg_ref[...] = ecard_ref[...][:, None] * m


def _tc_edge(part, ednm_p, ecard, b1, w10):
  return pl.pallas_call(
      _tc_edge_body,
      out_shape=jax.ShapeDtypeStruct((NE, HID), _f32),
  )(part, ednm_p, ecard, b1, w10)


def _tc_node_body(part_ref, ndnm_ref, ncard_ref, b0_ref, w01_ref, msg_ref):
  agg = part_ref[0, :NN, :] + part_ref[1, :NN, :]
  left0 = 1.0 / jnp.sum(ndnm_ref[...], axis=0)
  x0n = jnp.maximum(left0[:, None] * agg + b0_ref[...][None, :], 0.0)
  m = jnp.dot(x0n, w01_ref[...], preferred_element_type=_f32)
  msg_ref[...] = ncard_ref[...][:, None] * m


def _tc_node(part, ndnm_p, ncard, b0, w01):
  return pl.pallas_call(
      _tc_node_body,
      out_shape=jax.ShapeDtypeStruct((NN, HID), _f32),
  )(part, ndnm_p, ncard, b0, w01)


def _tc_final_body(part_ref, ndnm_ref, b0_ref, wlin_ref, blin_ref, out_ref):
  agg = part_ref[0, :NN, :] + part_ref[1, :NN, :]
  left0 = 1.0 / jnp.sum(ndnm_ref[...], axis=0)
  x = jnp.maximum(left0[:, None] * agg + b0_ref[...][None, :], 0.0)
  pooled = jnp.max(x, axis=0)
  out = jnp.dot(pooled[None, :], wlin_ref[...], preferred_element_type=_f32)
  out_ref[...] = out[0] + blin_ref[...]


def _tc_final(part, ndnm_p, b0, wlin, blin):
  return pl.pallas_call(
      _tc_final_body,
      out_shape=jax.ShapeDtypeStruct((1,), _f32),
  )(part, ndnm_p, b0, wlin, blin)


def kernel(x_0, node_idx, hyperedge_idx, W01_0, W10_0, b1_0, b0_0,
           W01_1, W10_1, b1_1, b0_1, W_lin, b_lin):
  nidx = node_idx.astype(_i32)
  eidx = hyperedge_idx.astype(_i32)

  ncnt_p, ecnt_p = _sc_counts(nidx, eidx)
  ncard, ecard, msg = _tc_prep(ncnt_p, ecnt_p, x_0, W01_0)
  ndnm_p, ednm_p = _sc_denoms(nidx, eidx, ncard, ecard)

  part_e = _rowpass_edge(msg, nidx, eidx)
  msg10 = _tc_edge(part_e, ednm_p, ecard, b1_0, W10_0)
  part_n = _rowpass_node(msg10, eidx, nidx)
  msg01b = _tc_node(part_n, ndnm_p, ncard, b0_0, W01_1)

  part_e2 = _rowpass_edge(msg01b, nidx, eidx)
  msg10b = _tc_edge(part_e2, ednm_p, ecard, b1_1, W10_1)
  part_n2 = _rowpass_node(msg10b, eidx, nidx)

  return _tc_final(part_n2, ndnm_p, b0_1, W_lin, b_lin)


# speedup vs baseline: 10.2392x; 10.2392x over previous
import functools

import jax
import jax.numpy as jnp
from jax import lax
from jax.experimental import pallas as pl
from jax.experimental.pallas import tpu as pltpu
from jax.experimental.pallas import tpu_sc as plsc

NN = 10000
NE = 5000
NI = 320000
HID = 128

NC = 2
NS = 16
NW = NC * NS
PER_W = NI // NW
G = 80
NCH = PER_W // G

NE_H = 5008
NE_PAD = 5120
NN_PAD = 10240

_MESH = plsc.VectorSubcoreMesh(
    core_axis_name="c", subcore_axis_name="s", num_cores=NC, num_subcores=NS)

_f32 = jnp.float32
_i32 = jnp.int32


def _zero_1d(ref, n16):
  z = jnp.zeros((16,), _f32)
  def body(i, _):
    ref[pl.ds(i * 16, 16)] = z
    return 0
  lax.fori_loop(0, n16, body, 0)


@functools.partial(
    pl.kernel,
    out_type=(jax.ShapeDtypeStruct((NW, NN), _f32),
              jax.ShapeDtypeStruct((NW, NE_H), _f32)),
    mesh=_MESH,
    compiler_params=pltpu.CompilerParams(needs_layout_passes=False),
    scratch_types=[
        pltpu.VMEM((PER_W,), _i32),
        pltpu.VMEM((PER_W,), _i32),
        pltpu.VMEM((NN,), _f32),
        pltpu.VMEM((NE_H,), _f32),
    ])
def _sc_counts(nidx_hbm, eidx_hbm, ncnt_out, ecnt_out,
               nidx_v, eidx_v, ncnt_v, ecnt_v):
  cid = lax.axis_index("c")
  sid = lax.axis_index("s")
  wid = cid * NS + sid
  base = wid * PER_W
  pltpu.sync_copy(nidx_hbm.at[pl.ds(base, PER_W)], nidx_v)
  pltpu.sync_copy(eidx_hbm.at[pl.ds(base, PER_W)], eidx_v)
  _zero_1d(ncnt_v, NN // 16)
  _zero_1d(ecnt_v, NE_H // 16)
  ones = jnp.ones((16,), _f32)
  def body(i, _):
    ni = nidx_v[pl.ds(i * 16, 16)]
    ei = eidx_v[pl.ds(i * 16, 16)]
    plsc.addupdate_scatter(ncnt_v, [ni], ones)
    plsc.addupdate_scatter(ecnt_v, [ei], ones)
    return 0
  lax.fori_loop(0, PER_W // 16, body, 0)
  pltpu.sync_copy(ncnt_v, ncnt_out.at[wid])
  pltpu.sync_copy(ecnt_v, ecnt_out.at[wid])


@functools.partial(
    pl.kernel,
    out_type=(jax.ShapeDtypeStruct((NW, NN), _f32),
              jax.ShapeDtypeStruct((NW, NE_H), _f32)),
    mesh=_MESH,
    compiler_params=pltpu.CompilerParams(needs_layout_passes=False),
    scratch_types=[
        pltpu.VMEM((PER_W,), _i32),
        pltpu.VMEM((PER_W,), _i32),
        pltpu.VMEM((NN,), _f32),
        pltpu.VMEM((NE,), _f32),
        pltpu.VMEM((NN,), _f32),
        pltpu.VMEM((NE_H,), _f32),
    ])
def _sc_denoms(nidx_hbm, eidx_hbm, ncard_hbm, ecard_hbm, ndnm_out, ednm_out,
               nidx_v, eidx_v, ncard_v, ecard_v, ndnm_v, ednm_v):
  cid = lax.axis_index("c")
  sid = lax.axis_index("s")
  wid = cid * NS + sid
  base = wid * PER_W
  pltpu.sync_copy(nidx_hbm.at[pl.ds(base, PER_W)], nidx_v)
  pltpu.sync_copy(eidx_hbm.at[pl.ds(base, PER_W)], eidx_v)
  pltpu.sync_copy(ncard_hbm, ncard_v)
  pltpu.sync_copy(ecard_hbm, ecard_v)
  _zero_1d(ndnm_v, NN // 16)
  _zero_1d(ednm_v, NE_H // 16)
  def body(i, _):
    ni = nidx_v[pl.ds(i * 16, 16)]
    ei = eidx_v[pl.ds(i * 16, 16)]
    nc = plsc.load_gather(ncard_v, [ni])
    ec = plsc.load_gather(ecard_v, [ei])
    plsc.addupdate_scatter(ednm_v, [ei], nc)
    plsc.addupdate_scatter(ndnm_v, [ni], ec)
    return 0
  lax.fori_loop(0, PER_W // 16, body, 0)
  pltpu.sync_copy(ndnm_v, ndnm_out.at[wid])
  pltpu.sync_copy(ednm_v, ednm_out.at[wid])


def _make_rowpass(s_pad):
  zr = s_pad // NS
  assert zr % G == 0

  @functools.partial(
      pl.kernel,
      out_type=jax.ShapeDtypeStruct((NC, s_pad, HID), _f32),
      mesh=_MESH,
      compiler_params=pltpu.CompilerParams(needs_layout_passes=False),
      scratch_types=[
          pltpu.VMEM((G,), _i32),
          pltpu.VMEM((G,), _i32),
          pltpu.VMEM((G, HID), _f32),
          pltpu.VMEM_SHARED((s_pad, HID), _f32),
          pltpu.SemaphoreType.DMA,
      ])
  def rowpass(table_hbm, gidx_hbm, sidx_hbm, out_hbm,
              gi_v, si_v, rows_v, acc_sh, sem):
    cid = lax.axis_index("c")
    sid = lax.axis_index("s")
    wid = cid * NS + sid
    z = jnp.zeros((16,), _f32)
    def zb(r, _):
      for j in range(HID // 16):
        rows_v[r, pl.ds(j * 16, 16)] = z
      return 0
    lax.fori_loop(0, G, zb, 0)
    for k in range(zr // G):
      pltpu.sync_copy(rows_v, acc_sh.at[pl.ds(sid * zr + k * G, G)])
    plsc.subcore_barrier()
    base = wid * PER_W
    def body(j, _):
      pltpu.sync_copy(gidx_hbm.at[pl.ds(base + j * G, G)], gi_v)
      pltpu.sync_copy(sidx_hbm.at[pl.ds(base + j * G, G)], si_v)
      pltpu.async_copy(table_hbm.at[gi_v], rows_v, sem).wait()
      pltpu.sync_copy(rows_v, acc_sh.at[si_v], add=True)
      return 0
    lax.fori_loop(0, NCH, body, 0)
    plsc.subcore_barrier()
    for k in range(zr // G):
      r0 = sid * zr + k * G
      pltpu.sync_copy(acc_sh.at[pl.ds(r0, G)], rows_v)
      pltpu.sync_copy(rows_v, out_hbm.at[cid, pl.ds(r0, G)])

  return rowpass


_rowpass_edge = _make_rowpass(NE_PAD)
_rowpass_node = _make_rowpass(NN_PAD)


def _tc_prep_body(ncnt_ref, ecnt_ref, x0_ref, w01_ref,
                  ncard_ref, ecard_ref, msg_ref):
  ncnt = jnp.sum(ncnt_ref[...], axis=0)
  ecnt = jnp.sum(ecnt_ref[...], axis=0)[:NE]
  ncard = lax.rsqrt(ncnt)
  r = lax.rsqrt(ecnt)
  ecard = r * r * r
  ncard_ref[...] = ncard
  ecard_ref[...] = ecard
  m = jnp.dot(x0_ref[...], w01_ref[...], preferred_element_type=_f32)
  msg_ref[...] = ncard[:, None] * m


def _tc_prep(ncnt_p, ecnt_p, x0, w01):
  return pl.pallas_call(
      _tc_prep_body,
      out_shape=(jax.ShapeDtypeStruct((NN,), _f32),
                 jax.ShapeDtypeStruct((NE,), _f32),
                 jax.ShapeDtypeStruct((NN, HID), _f32)),
  )(ncnt_p, ecnt_p, x0, w01)


def _tc_edge_body(part_ref, ednm_ref, ecard_ref, b1_ref, w10_ref, msg_ref):
  agg = part_ref[0, :NE, :] + part_ref[1, :NE, :]
  left1 = 1.0 / jnp.sum(ednm_ref[...], axis=0)[:NE]
  x1 = jnp.maximum(left1[:, None] * agg + b1_ref[...][None, :], 0.0)
  m = jnp.dot(x1, w10_ref[...], preferred_element_type=_f32)
  msg_ref[...] = ecard_ref[...][:, None] * m


def _tc_edge(part, ednm_p, ecard, b1, w10):
  return pl.pallas_call(
      _tc_edge_body,
      out_shape=jax.ShapeDtypeStruct((NE, HID), _f32),
  )(part, ednm_p, ecard, b1, w10)


def _tc_node_body(part_ref, ndnm_ref, ncard_ref, b0_ref, w01_ref, msg_ref):
  agg = part_ref[0, :NN, :] + part_ref[1, :NN, :]
  left0 = 1.0 / jnp.sum(ndnm_ref[...], axis=0)
  x0n = jnp.maximum(left0[:, None] * agg + b0_ref[...][None, :], 0.0)
  m = jnp.dot(x0n, w01_ref[...], preferred_element_type=_f32)
  msg_ref[...] = ncard_ref[...][:, None] * m


def _tc_node(part, ndnm_p, ncard, b0, w01):
  return pl.pallas_call(
      _tc_node_body,
      out_shape=jax.ShapeDtypeStruct((NN, HID), _f32),
  )(part, ndnm_p, ncard, b0, w01)


def _tc_final_body(part_ref, ndnm_ref, b0_ref, wlin_ref, blin_ref, out_ref):
  agg = part_ref[0, :NN, :] + part_ref[1, :NN, :]
  left0 = 1.0 / jnp.sum(ndnm_ref[...], axis=0)
  x = jnp.maximum(left0[:, None] * agg + b0_ref[...][None, :], 0.0)
  pooled = jnp.max(x, axis=0)
  out = jnp.dot(pooled[None, :], wlin_ref[...], preferred_element_type=_f32)
  out_ref[...] = out[0] + blin_ref[...]


def _tc_final(part, ndnm_p, b0, wlin, blin):
  return pl.pallas_call(
      _tc_final_body,
      out_shape=jax.ShapeDtypeStruct((1,), _f32),
  )(part, ndnm_p, b0, wlin, blin)


def kernel(x_0, node_idx, hyperedge_idx, W01_0, W10_0, b1_0, b0_0,
           W01_1, W10_1, b1_1, b0_1, W_lin, b_lin):
  nidx = node_idx.astype(_i32)
  eidx = hyperedge_idx.astype(_i32)

  ncnt_p, ecnt_p = _sc_counts(nidx, eidx)
  ncard, ecard, msg = _tc_prep(ncnt_p, ecnt_p, x_0, W01_0)
  ndnm_p, ednm_p = _sc_denoms(nidx, eidx, ncard, ecard)

  part_e = _rowpass_edge(msg, nidx, eidx)
  msg10 = _tc_edge(part_e, ednm_p, ecard, b1_0, W10_0)
  part_n = _rowpass_node(msg10, eidx, nidx)
  msg01b = _tc_node(part_n, ndnm_p, ncard, b0_0, W01_1)

  part_e2 = _rowpass_edge(msg01b, nidx, eidx)
  msg10b = _tc_edge(part_e2, ednm_p, ecard, b1_1, W10_1)
  part_n2 = _rowpass_node(msg10b, eidx, nidx)

  return _tc_final(part_n2, ndnm_p, b0_1, W_lin, b_lin)


# speedup vs baseline: 23.1439x; 2.2603x over previous
import functools

import jax
import jax.numpy as jnp
from jax import lax
from jax.experimental import pallas as pl
from jax.experimental.pallas import tpu as pltpu
from jax.experimental.pallas import tpu_sc as plsc

NN = 10000
NE = 5000
NI = 320000
HID = 128

NC = 2
NS = 16
NW = NC * NS
PER_W = NI // NW
G = 80
NCH = PER_W // G

NE_H = 5008
NE_PAD = 5120
NN_PAD = 10240

_MESH = plsc.VectorSubcoreMesh(
    core_axis_name="c", subcore_axis_name="s", num_cores=NC, num_subcores=NS)

_f32 = jnp.float32
_i32 = jnp.int32


def _zero_1d(ref, n16):
  z = jnp.zeros((16,), _f32)
  def body(i, _):
    ref[pl.ds(i * 16, 16)] = z
    return 0
  lax.fori_loop(0, n16, body, 0)


@functools.partial(
    pl.kernel,
    out_type=(jax.ShapeDtypeStruct((NW, NN), _f32),
              jax.ShapeDtypeStruct((NW, NE_H), _f32)),
    mesh=_MESH,
    compiler_params=pltpu.CompilerParams(needs_layout_passes=False),
    scratch_types=[
        pltpu.VMEM((PER_W,), _i32),
        pltpu.VMEM((PER_W,), _i32),
        pltpu.VMEM((NN,), _f32),
        pltpu.VMEM((NE_H,), _f32),
    ])
def _sc_counts(nidx_hbm, eidx_hbm, ncnt_out, ecnt_out,
               nidx_v, eidx_v, ncnt_v, ecnt_v):
  cid = lax.axis_index("c")
  sid = lax.axis_index("s")
  wid = cid * NS + sid
  base = wid * PER_W
  pltpu.sync_copy(nidx_hbm.at[pl.ds(base, PER_W)], nidx_v)
  pltpu.sync_copy(eidx_hbm.at[pl.ds(base, PER_W)], eidx_v)
  _zero_1d(ncnt_v, NN // 16)
  _zero_1d(ecnt_v, NE_H // 16)
  ones = jnp.ones((16,), _f32)
  def body(i, _):
    ni = nidx_v[pl.ds(i * 16, 16)]
    ei = eidx_v[pl.ds(i * 16, 16)]
    plsc.addupdate_scatter(ncnt_v, [ni], ones)
    plsc.addupdate_scatter(ecnt_v, [ei], ones)
    return 0
  lax.fori_loop(0, PER_W // 16, body, 0)
  pltpu.sync_copy(ncnt_v, ncnt_out.at[wid])
  pltpu.sync_copy(ecnt_v, ecnt_out.at[wid])


@functools.partial(
    pl.kernel,
    out_type=(jax.ShapeDtypeStruct((NW, NN), _f32),
              jax.ShapeDtypeStruct((NW, NE_H), _f32)),
    mesh=_MESH,
    compiler_params=pltpu.CompilerParams(needs_layout_passes=False),
    scratch_types=[
        pltpu.VMEM((PER_W,), _i32),
        pltpu.VMEM((PER_W,), _i32),
        pltpu.VMEM((NN,), _f32),
        pltpu.VMEM((NE,), _f32),
        pltpu.VMEM((NN,), _f32),
        pltpu.VMEM((NE_H,), _f32),
    ])
def _sc_denoms(nidx_hbm, eidx_hbm, ncard_hbm, ecard_hbm, ndnm_out, ednm_out,
               nidx_v, eidx_v, ncard_v, ecard_v, ndnm_v, ednm_v):
  cid = lax.axis_index("c")
  sid = lax.axis_index("s")
  wid = cid * NS + sid
  base = wid * PER_W
  pltpu.sync_copy(nidx_hbm.at[pl.ds(base, PER_W)], nidx_v)
  pltpu.sync_copy(eidx_hbm.at[pl.ds(base, PER_W)], eidx_v)
  pltpu.sync_copy(ncard_hbm, ncard_v)
  pltpu.sync_copy(ecard_hbm, ecard_v)
  _zero_1d(ndnm_v, NN // 16)
  _zero_1d(ednm_v, NE_H // 16)
  def body(i, _):
    ni = nidx_v[pl.ds(i * 16, 16)]
    ei = eidx_v[pl.ds(i * 16, 16)]
    nc = plsc.load_gather(ncard_v, [ni])
    ec = plsc.load_gather(ecard_v, [ei])
    plsc.addupdate_scatter(ednm_v, [ei], nc)
    plsc.addupdate_scatter(ndnm_v, [ni], ec)
    return 0
  lax.fori_loop(0, PER_W // 16, body, 0)
  pltpu.sync_copy(ndnm_v, ndnm_out.at[wid])
  pltpu.sync_copy(ednm_v, ednm_out.at[wid])


NB = 5


def _make_rowpass(s_pad, g, nb):
  zr = s_pad // NS
  nch = PER_W // g
  assert zr % g == 0 and nch % nb == 0 and PER_W % g == 0

  @functools.partial(
      pl.kernel,
      out_type=jax.ShapeDtypeStruct((NC, s_pad, HID), _f32),
      mesh=_MESH,
      compiler_params=pltpu.CompilerParams(needs_layout_passes=False),
      scratch_types=[
          pltpu.VMEM((PER_W,), _i32),
          pltpu.VMEM((PER_W,), _i32),
          pltpu.VMEM((nb, g, HID), _f32),
          pltpu.VMEM_SHARED((s_pad, HID), _f32),
          [pltpu.SemaphoreType.DMA] * nb,
      ])
  def rowpass(table_hbm, gidx_hbm, sidx_hbm, out_hbm,
              gi_v, si_v, rows_v, acc_sh, sems):
    cid = lax.axis_index("c")
    sid = lax.axis_index("s")
    wid = cid * NS + sid
    base = wid * PER_W
    pltpu.sync_copy(gidx_hbm.at[pl.ds(base, PER_W)], gi_v)
    pltpu.sync_copy(sidx_hbm.at[pl.ds(base, PER_W)], si_v)
    z = jnp.zeros((16,), _f32)
    def zb(r, _):
      for j in range(HID // 16):
        rows_v[0, r, pl.ds(j * 16, 16)] = z
      return 0
    lax.fori_loop(0, g, zb, 0)
    for k in range(zr // g):
      pltpu.sync_copy(rows_v.at[0], acc_sh.at[pl.ds(sid * zr + k * g, g)])
    plsc.subcore_barrier()
    def gidx_chunk(j):
      return gi_v.at[pl.ds(j * g, g)]
    def sidx_chunk(j):
      return si_v.at[pl.ds(j * g, g)]
    for b in range(nb):
      pltpu.async_copy(table_hbm.at[gidx_chunk(b)], rows_v.at[b], sems[b])
    def body(jo, _):
      for b in range(nb):
        j = jo * nb + b
        pltpu.make_async_copy(table_hbm.at[gidx_chunk(0)],
                              rows_v.at[b], sems[b]).wait()
        pltpu.sync_copy(rows_v.at[b], acc_sh.at[sidx_chunk(j)], add=True)
        pltpu.async_copy(table_hbm.at[gidx_chunk(j + nb)], rows_v.at[b],
                         sems[b])
      return 0
    lax.fori_loop(0, nch // nb - 1, body, 0)
    for b in range(nb):
      j = nch - nb + b
      pltpu.make_async_copy(table_hbm.at[gidx_chunk(0)],
                            rows_v.at[b], sems[b]).wait()
      pltpu.sync_copy(rows_v.at[b], acc_sh.at[sidx_chunk(j)], add=True)
    plsc.subcore_barrier()
    for k in range(zr // g):
      r0 = sid * zr + k * g
      pltpu.sync_copy(acc_sh.at[pl.ds(r0, g)], rows_v.at[0])
      pltpu.sync_copy(rows_v.at[0], out_hbm.at[cid, pl.ds(r0, g)])

  return rowpass


_rowpass_edge = _make_rowpass(NE_PAD, G, NB)


G2 = 40
NCH2 = PER_W // G2


_rowpass_node = _make_rowpass(NN_PAD, G2, 2)


def _tc_prep_body(ncnt_ref, ecnt_ref, x0_ref, w01_ref,
                  ncard_ref, ecard_ref, msg_ref):
  ncnt = jnp.sum(ncnt_ref[...], axis=0)
  ecnt = jnp.sum(ecnt_ref[...], axis=0)[:NE]
  ncard = lax.rsqrt(ncnt)
  r = lax.rsqrt(ecnt)
  ecard = r * r * r
  ncard_ref[...] = ncard
  ecard_ref[...] = ecard
  m = jnp.dot(x0_ref[...], w01_ref[...], preferred_element_type=_f32)
  msg_ref[...] = ncard[:, None] * m


def _tc_prep(ncnt_p, ecnt_p, x0, w01):
  return pl.pallas_call(
      _tc_prep_body,
      out_shape=(jax.ShapeDtypeStruct((NN,), _f32),
                 jax.ShapeDtypeStruct((NE,), _f32),
                 jax.ShapeDtypeStruct((NN, HID), _f32)),
  )(ncnt_p, ecnt_p, x0, w01)


def _tc_edge_body(part_ref, ednm_ref, ecard_ref, b1_ref, w10_ref, msg_ref):
  agg = part_ref[0, :NE, :] + part_ref[1, :NE, :]
  left1 = 1.0 / jnp.sum(ednm_ref[...], axis=0)[:NE]
  x1 = jnp.maximum(left1[:, None] * agg + b1_ref[...][None, :], 0.0)
  m = jnp.dot(x1, w10_ref[...], preferred_element_type=_f32)
  msg_ref[...] = ecard_ref[...][:, None] * m


def _tc_edge(part, ednm_p, ecard, b1, w10):
  return pl.pallas_call(
      _tc_edge_body,
      out_shape=jax.ShapeDtypeStruct((NE, HID), _f32),
  )(part, ednm_p, ecard, b1, w10)


def _tc_node_body(part_ref, ndnm_ref, ncard_ref, b0_ref, w01_ref, msg_ref):
  agg = part_ref[0, :NN, :] + part_ref[1, :NN, :]
  left0 = 1.0 / jnp.sum(ndnm_ref[...], axis=0)
  x0n = jnp.maximum(left0[:, None] * agg + b0_ref[...][None, :], 0.0)
  m = jnp.dot(x0n, w01_ref[...], preferred_element_type=_f32)
  msg_ref[...] = ncard_ref[...][:, None] * m


def _tc_node(part, ndnm_p, ncard, b0, w01):
  return pl.pallas_call(
      _tc_node_body,
      out_shape=jax.ShapeDtypeStruct((NN, HID), _f32),
  )(part, ndnm_p, ncard, b0, w01)


def _tc_final_body(part_ref, ndnm_ref, b0_ref, wlin_ref, blin_ref, out_ref):
  agg = part_ref[0, :NN, :] + part_ref[1, :NN, :]
  left0 = 1.0 / jnp.sum(ndnm_ref[...], axis=0)
  x = jnp.maximum(left0[:, None] * agg + b0_ref[...][None, :], 0.0)
  pooled = jnp.max(x, axis=0)
  out = jnp.dot(pooled[None, :], wlin_ref[...], preferred_element_type=_f32)
  out_ref[...] = out[0] + blin_ref[...]


def _tc_final(part, ndnm_p, b0, wlin, blin):
  return pl.pallas_call(
      _tc_final_body,
      out_shape=jax.ShapeDtypeStruct((1,), _f32),
  )(part, ndnm_p, b0, wlin, blin)


def kernel(x_0, node_idx, hyperedge_idx, W01_0, W10_0, b1_0, b0_0,
           W01_1, W10_1, b1_1, b0_1, W_lin, b_lin):
  nidx = node_idx.astype(_i32)
  eidx = hyperedge_idx.astype(_i32)

  ncnt_p, ecnt_p = _sc_counts(nidx, eidx)
  ncard, ecard, msg = _tc_prep(ncnt_p, ecnt_p, x_0, W01_0)
  ndnm_p, ednm_p = _sc_denoms(nidx, eidx, ncard, ecard)

  part_e = _rowpass_edge(msg, nidx, eidx)
  msg10 = _tc_edge(part_e, ednm_p, ecard, b1_0, W10_0)
  part_n = _rowpass_node(msg10, eidx, nidx)
  msg01b = _tc_node(part_n, ndnm_p, ncard, b0_0, W01_1)

  part_e2 = _rowpass_edge(msg01b, nidx, eidx)
  msg10b = _tc_edge(part_e2, ednm_p, ecard, b1_1, W10_1)
  part_n2 = _rowpass_node(msg10b, eidx, nidx)

  return _tc_final(part_n2, ndnm_p, b0_1, W_lin, b_lin)


# speedup vs baseline: 27.0490x; 1.1687x over previous
import functools

import jax
import jax.numpy as jnp
from jax import lax
from jax.experimental import pallas as pl
from jax.experimental.pallas import tpu as pltpu
from jax.experimental.pallas import tpu_sc as plsc

NN = 10000
NE = 5000
NI = 320000
HID = 128

NC = 2
NS = 16
NW = NC * NS
PER_W = NI // NW
G = 80
NCH = PER_W // G

NE_H = 5008
NE_PAD = 5120
NN_PAD = 10240

_MESH = plsc.VectorSubcoreMesh(
    core_axis_name="c", subcore_axis_name="s", num_cores=NC, num_subcores=NS)

_f32 = jnp.float32
_i32 = jnp.int32


def _zero_1d(ref, n16):
  z = jnp.zeros((16,), _f32)
  def body(i, _):
    ref[pl.ds(i * 16, 16)] = z
    return 0
  lax.fori_loop(0, n16, body, 0)


@functools.partial(
    pl.kernel,
    out_type=(jax.ShapeDtypeStruct((NW, NN), _f32),
              jax.ShapeDtypeStruct((NW, NE_H), _f32)),
    mesh=_MESH,
    compiler_params=pltpu.CompilerParams(needs_layout_passes=False),
    scratch_types=[
        pltpu.VMEM((PER_W,), _i32),
        pltpu.VMEM((PER_W,), _i32),
        pltpu.VMEM((NN,), _f32),
        pltpu.VMEM((NE_H,), _f32),
    ])
def _sc_counts(nidx_hbm, eidx_hbm, ncnt_out, ecnt_out,
               nidx_v, eidx_v, ncnt_v, ecnt_v):
  cid = lax.axis_index("c")
  sid = lax.axis_index("s")
  wid = cid * NS + sid
  base = wid * PER_W
  pltpu.sync_copy(nidx_hbm.at[pl.ds(base, PER_W)], nidx_v)
  pltpu.sync_copy(eidx_hbm.at[pl.ds(base, PER_W)], eidx_v)
  _zero_1d(ncnt_v, NN // 16)
  _zero_1d(ecnt_v, NE_H // 16)
  ones = jnp.ones((16,), _f32)
  def body(i, _):
    ni = nidx_v[pl.ds(i * 16, 16)]
    ei = eidx_v[pl.ds(i * 16, 16)]
    plsc.addupdate_scatter(ncnt_v, [ni], ones)
    plsc.addupdate_scatter(ecnt_v, [ei], ones)
    return 0
  lax.fori_loop(0, PER_W // 16, body, 0)
  pltpu.sync_copy(ncnt_v, ncnt_out.at[wid])
  pltpu.sync_copy(ecnt_v, ecnt_out.at[wid])


@functools.partial(
    pl.kernel,
    out_type=(jax.ShapeDtypeStruct((NW, NN), _f32),
              jax.ShapeDtypeStruct((NW, NE_H), _f32)),
    mesh=_MESH,
    compiler_params=pltpu.CompilerParams(needs_layout_passes=False),
    scratch_types=[
        pltpu.VMEM((PER_W,), _i32),
        pltpu.VMEM((PER_W,), _i32),
        pltpu.VMEM((NN,), _f32),
        pltpu.VMEM((NE,), _f32),
        pltpu.VMEM((NN,), _f32),
        pltpu.VMEM((NE_H,), _f32),
    ])
def _sc_denoms(nidx_hbm, eidx_hbm, ncard_hbm, ecard_hbm, ndnm_out, ednm_out,
               nidx_v, eidx_v, ncard_v, ecard_v, ndnm_v, ednm_v):
  cid = lax.axis_index("c")
  sid = lax.axis_index("s")
  wid = cid * NS + sid
  base = wid * PER_W
  pltpu.sync_copy(nidx_hbm.at[pl.ds(base, PER_W)], nidx_v)
  pltpu.sync_copy(eidx_hbm.at[pl.ds(base, PER_W)], eidx_v)
  pltpu.sync_copy(ncard_hbm, ncard_v)
  pltpu.sync_copy(ecard_hbm, ecard_v)
  _zero_1d(ndnm_v, NN // 16)
  _zero_1d(ednm_v, NE_H // 16)
  def body(i, _):
    ni = nidx_v[pl.ds(i * 16, 16)]
    ei = eidx_v[pl.ds(i * 16, 16)]
    nc = plsc.load_gather(ncard_v, [ni])
    ec = plsc.load_gather(ecard_v, [ei])
    plsc.addupdate_scatter(ednm_v, [ei], nc)
    plsc.addupdate_scatter(ndnm_v, [ni], ec)
    return 0
  lax.fori_loop(0, PER_W // 16, body, 0)
  pltpu.sync_copy(ndnm_v, ndnm_out.at[wid])
  pltpu.sync_copy(ednm_v, ednm_out.at[wid])


NB = 5


def _make_rowpass(s_pad, g, nb):
  zr = s_pad // NS
  nch = PER_W // g
  ng = (nch - nb) // nb
  assert zr % g == 0 and PER_W % g == 0 and ng >= 1

  @functools.partial(
      pl.kernel,
      out_type=jax.ShapeDtypeStruct((NC, s_pad, HID), _f32),
      mesh=_MESH,
      compiler_params=pltpu.CompilerParams(needs_layout_passes=False),
      scratch_types=[
          pltpu.VMEM((PER_W,), _i32),
          pltpu.VMEM((PER_W,), _i32),
          pltpu.VMEM((nb, g, HID), _f32),
          pltpu.VMEM_SHARED((s_pad, HID), _f32),
          [pltpu.SemaphoreType.DMA] * nb,
      ])
  def rowpass(table_hbm, gidx_hbm, sidx_hbm, out_hbm,
              gi_v, si_v, rows_v, acc_sh, sems):
    cid = lax.axis_index("c")
    sid = lax.axis_index("s")
    wid = cid * NS + sid
    base = wid * PER_W
    pltpu.sync_copy(gidx_hbm.at[pl.ds(base, PER_W)], gi_v)
    pltpu.sync_copy(sidx_hbm.at[pl.ds(base, PER_W)], si_v)
    z = jnp.zeros((16,), _f32)
    def zb(r, _):
      for j in range(HID // 16):
        rows_v[0, r, pl.ds(j * 16, 16)] = z
      return 0
    lax.fori_loop(0, g, zb, 0)
    for k in range(zr // g):
      pltpu.sync_copy(rows_v.at[0], acc_sh.at[pl.ds(sid * zr + k * g, g)])
    plsc.subcore_barrier()
    def gidx_chunk(j):
      return gi_v.at[pl.ds(j * g, g)]
    def sidx_chunk(j):
      return si_v.at[pl.ds(j * g, g)]
    for b in range(nb):
      pltpu.async_copy(table_hbm.at[gidx_chunk(b)], rows_v.at[b], sems[b])
    def body(jo, _):
      for b in range(nb):
        j = jo * nb + b
        pltpu.make_async_copy(table_hbm.at[gidx_chunk(0)],
                              rows_v.at[b], sems[b]).wait()
        pltpu.sync_copy(rows_v.at[b], acc_sh.at[sidx_chunk(j)], add=True)
        pltpu.async_copy(table_hbm.at[gidx_chunk(j + nb)], rows_v.at[b],
                         sems[b])
      return 0
    lax.fori_loop(0, ng, body, 0)
    for j in range(ng * nb, nch):
      b = j % nb
      pltpu.make_async_copy(table_hbm.at[gidx_chunk(0)],
                            rows_v.at[b], sems[b]).wait()
      pltpu.sync_copy(rows_v.at[b], acc_sh.at[sidx_chunk(j)], add=True)
      if j + nb < nch:
        pltpu.async_copy(table_hbm.at[gidx_chunk(j + nb)], rows_v.at[b],
                         sems[b])
    plsc.subcore_barrier()
    for k in range(zr // g):
      r0 = sid * zr + k * g
      pltpu.sync_copy(acc_sh.at[pl.ds(r0, g)], rows_v.at[0])
      pltpu.sync_copy(rows_v.at[0], out_hbm.at[cid, pl.ds(r0, g)])

  return rowpass


_rowpass_edge = _make_rowpass(NE_PAD, G, NB)


_rowpass_node = _make_rowpass(NN_PAD, G, 2)


def _tc_prep_body(ncnt_ref, ecnt_ref, x0_ref, w01_ref,
                  ncard_ref, ecard_ref, msg_ref):
  ncnt = jnp.sum(ncnt_ref[...], axis=0)
  ecnt = jnp.sum(ecnt_ref[...], axis=0)[:NE]
  ncard = lax.rsqrt(ncnt)
  r = lax.rsqrt(ecnt)
  ecard = r * r * r
  ncard_ref[...] = ncard
  ecard_ref[...] = ecard
  m = jnp.dot(x0_ref[...], w01_ref[...], preferred_element_type=_f32)
  msg_ref[...] = ncard[:, None] * m


def _tc_prep(ncnt_p, ecnt_p, x0, w01):
  return pl.pallas_call(
      _tc_prep_body,
      out_shape=(jax.ShapeDtypeStruct((NN,), _f32),
                 jax.ShapeDtypeStruct((NE,), _f32),
                 jax.ShapeDtypeStruct((NN, HID), _f32)),
  )(ncnt_p, ecnt_p, x0, w01)


def _tc_edge_body(part_ref, ednm_ref, ecard_ref, b1_ref, w10_ref, msg_ref):
  agg = part_ref[0, :NE, :] + part_ref[1, :NE, :]
  left1 = 1.0 / jnp.sum(ednm_ref[...], axis=0)[:NE]
  x1 = jnp.maximum(left1[:, None] * agg + b1_ref[...][None, :], 0.0)
  m = jnp.dot(x1, w10_ref[...], preferred_element_type=_f32)
  msg_ref[...] = ecard_ref[...][:, None] * m


def _tc_edge(part, ednm_p, ecard, b1, w10):
  return pl.pallas_call(
      _tc_edge_body,
      out_shape=jax.ShapeDtypeStruct((NE, HID), _f32),
  )(part, ednm_p, ecard, b1, w10)


def _tc_node_body(part_ref, ndnm_ref, ncard_ref, b0_ref, w01_ref, msg_ref):
  agg = part_ref[0, :NN, :] + part_ref[1, :NN, :]
  left0 = 1.0 / jnp.sum(ndnm_ref[...], axis=0)
  x0n = jnp.maximum(left0[:, None] * agg + b0_ref[...][None, :], 0.0)
  m = jnp.dot(x0n, w01_ref[...], preferred_element_type=_f32)
  msg_ref[...] = ncard_ref[...][:, None] * m


def _tc_node(part, ndnm_p, ncard, b0, w01):
  return pl.pallas_call(
      _tc_node_body,
      out_shape=jax.ShapeDtypeStruct((NN, HID), _f32),
  )(part, ndnm_p, ncard, b0, w01)


def _tc_final_body(part_ref, ndnm_ref, b0_ref, wlin_ref, blin_ref, out_ref):
  agg = part_ref[0, :NN, :] + part_ref[1, :NN, :]
  left0 = 1.0 / jnp.sum(ndnm_ref[...], axis=0)
  x = jnp.maximum(left0[:, None] * agg + b0_ref[...][None, :], 0.0)
  pooled = jnp.max(x, axis=0)
  out = jnp.dot(pooled[None, :], wlin_ref[...], preferred_element_type=_f32)
  out_ref[...] = out[0] + blin_ref[...]


def _tc_final(part, ndnm_p, b0, wlin, blin):
  return pl.pallas_call(
      _tc_final_body,
      out_shape=jax.ShapeDtypeStruct((1,), _f32),
  )(part, ndnm_p, b0, wlin, blin)


def kernel(x_0, node_idx, hyperedge_idx, W01_0, W10_0, b1_0, b0_0,
           W01_1, W10_1, b1_1, b0_1, W_lin, b_lin):
  nidx = node_idx.astype(_i32)
  eidx = hyperedge_idx.astype(_i32)

  ncnt_p, ecnt_p = _sc_counts(nidx, eidx)
  ncard, ecard, msg = _tc_prep(ncnt_p, ecnt_p, x_0, W01_0)
  ndnm_p, ednm_p = _sc_denoms(nidx, eidx, ncard, ecard)

  part_e = _rowpass_edge(msg, nidx, eidx)
  msg10 = _tc_edge(part_e, ednm_p, ecard, b1_0, W10_0)
  part_n = _rowpass_node(msg10, eidx, nidx)
  msg01b = _tc_node(part_n, ndnm_p, ncard, b0_0, W01_1)

  part_e2 = _rowpass_edge(msg01b, nidx, eidx)
  msg10b = _tc_edge(part_e2, ednm_p, ecard, b1_1, W10_1)
  part_n2 = _rowpass_node(msg10b, eidx, nidx)

  return _tc_final(part_n2, ndnm_p, b0_1, W_lin, b_lin)
